# baseline (speedup 1.0000x reference)
import math

import jax
import jax.numpy as jnp
from jax.experimental import pallas as pl
from jax.experimental.pallas import tpu as pltpu

_SQRT_2_OVER_PI = math.sqrt(2.0 / math.pi)


def _gelu_block(x_ref, o_ref):
    x = x_ref[...]
    inner = _SQRT_2_OVER_PI * (x + 0.044715 * (x * x * x))
    o_ref[...] = 0.5 * x * (1.0 + jnp.tanh(inner))


def kernel(x, log_k):
    B, T, D = x.shape
    rows = B * T
    x2 = x.reshape(rows, D)
    block_rows = 1024
    grid = (rows // block_rows,)
    y2 = pl.pallas_call(
        _gelu_block,
        grid=grid,
        in_specs=[pl.BlockSpec((block_rows, D), lambda i: (i, 0))],
        out_specs=pl.BlockSpec((block_rows, D), lambda i: (i, 0)),
        out_shape=jax.ShapeDtypeStruct((rows, D), x.dtype),
        compiler_params=pltpu.CompilerParams(
            dimension_semantics=("arbitrary",),
        ),
    )(x2)
    return y2.reshape(B, T, D)
